# baseline (speedup 1.0000x reference)
import numpy as np
import jax
import jax.numpy as jnp
from jax import lax
from jax.experimental import pallas as pl
from jax.experimental.pallas import tpu as pltpu
from jax.experimental.pallas import tpu_sc as plsc

_B, _N = 4096, 128
_M = _N * _N
_K = _N * (_N + 1) // 2

_NW = 32
_BPW = _B // _NW

_ROW_IDX, _COL_IDX = np.triu_indices(_N)
_FLAT_IDX = (_ROW_IDX * _N + _COL_IDX).astype(np.int32)


def _sc_body(x_hbm, idx_hbm, out_hbm, idx_v, in_v, out_v, sem):
    c = lax.axis_index("c")
    s = lax.axis_index("s")
    wid = s * 2 + c
    b0 = wid * _BPW

    pltpu.sync_copy(idx_hbm, idx_v)

    def row(r, carry):
        b = b0 + r
        pltpu.sync_copy(x_hbm.at[b], in_v)

        def chunk(k, c2):
            o = k * 16
            idx = idx_v[pl.ds(o, 16)]
            vals = plsc.load_gather(in_v, [idx])
            out_v[pl.ds(o, 16)] = vals
            return c2

        lax.fori_loop(0, _K // 16, chunk, 0)
        pltpu.sync_copy(out_v, out_hbm.at[b])
        return carry

    lax.fori_loop(0, _BPW, row, 0)


def kernel(input):
    x2 = input.reshape(_B, _M)
    fidx = jnp.asarray(_FLAT_IDX)
    mesh = plsc.VectorSubcoreMesh(core_axis_name="c", subcore_axis_name="s")
    f = pl.kernel(
        _sc_body,
        mesh=mesh,
        out_type=jax.ShapeDtypeStruct((_B, _K), jnp.float32),
        scratch_types=[
            pltpu.VMEM((_K,), jnp.int32),
            pltpu.VMEM((_M,), jnp.float32),
            pltpu.VMEM((_K,), jnp.float32),
            pltpu.SemaphoreType.DMA,
        ],
        compiler_params=pltpu.CompilerParams(
            use_tc_tiling_on_sc=False, needs_layout_passes=False
        ),
    )
    return f(x2, fidx)


# speedup vs baseline: 1.6900x; 1.6900x over previous
import numpy as np
import jax
import jax.numpy as jnp
from jax import lax
from jax.experimental import pallas as pl
from jax.experimental.pallas import tpu as pltpu
from jax.experimental.pallas import tpu_sc as plsc

_B, _N = 4096, 128
_M = _N * _N
_K = _N * (_N + 1) // 2
_NT = _K // 16
_UNROLL = 12

_NW = 32
_BPW = _B // _NW

_ROW_IDX, _COL_IDX = np.triu_indices(_N)
_FLAT_IDX = (_ROW_IDX * _N + _COL_IDX).astype(np.int32)


def _sc_body(x_hbm, idx_hbm, out_hbm, idx_v, in0, in1, ou0, ou1,
             is0, is1, os0, os1):
    c = lax.axis_index("c")
    s = lax.axis_index("s")
    wid = s * 2 + c
    b0 = wid * _BPW

    pltpu.sync_copy(idx_hbm, idx_v)

    bufs = ((in0, ou0, is0, os0), (in1, ou1, is1, os1))

    def start_in(p, b):
        iv, _, isem, _ = bufs[p]
        pltpu.async_copy(x_hbm.at[b], iv, isem)

    def wait_in(p, b):
        iv, _, isem, _ = bufs[p]
        pltpu.make_async_copy(x_hbm.at[b], iv, isem).wait()

    def start_out(p, b):
        _, ov, _, osem = bufs[p]
        pltpu.async_copy(ov, out_hbm.at[b], osem)

    def wait_out(p, b):
        _, ov, _, osem = bufs[p]
        pltpu.make_async_copy(ov, out_hbm.at[b], osem).wait()

    start_in(0, b0)
    start_in(1, b0 + 1)

    def pair(rr, carry):
        for p in (0, 1):
            r = rr * 2 + p
            b = b0 + r
            iv, ov, _, _ = bufs[p]
            wait_in(p, b)

            @pl.when(rr > 0)
            def _():
                wait_out(p, b - 2)

            def chunk(k, c2):
                base = k * (16 * _UNROLL)
                for u in range(_UNROLL):
                    o = base + u * 16
                    idx = idx_v[pl.ds(o, 16)]
                    ov[pl.ds(o, 16)] = plsc.load_gather(iv, [idx])
                return c2

            lax.fori_loop(0, _NT // _UNROLL, chunk, 0)
            start_out(p, b)

            @pl.when(r + 2 < _BPW)
            def _():
                start_in(p, b + 2)
        return carry

    lax.fori_loop(0, _BPW // 2, pair, 0)

    wait_out(0, b0 + _BPW - 2)
    wait_out(1, b0 + _BPW - 1)


def kernel(input):
    x2 = input.reshape(_B, _M)
    fidx = jnp.asarray(_FLAT_IDX)
    mesh = plsc.VectorSubcoreMesh(core_axis_name="c", subcore_axis_name="s")
    f = pl.kernel(
        _sc_body,
        mesh=mesh,
        out_type=jax.ShapeDtypeStruct((_B, _K), jnp.float32),
        scratch_types=[
            pltpu.VMEM((_K,), jnp.int32),
            pltpu.VMEM((_M,), jnp.float32),
            pltpu.VMEM((_M,), jnp.float32),
            pltpu.VMEM((_K,), jnp.float32),
            pltpu.VMEM((_K,), jnp.float32),
            pltpu.SemaphoreType.DMA,
            pltpu.SemaphoreType.DMA,
            pltpu.SemaphoreType.DMA,
            pltpu.SemaphoreType.DMA,
        ],
        compiler_params=pltpu.CompilerParams(
            use_tc_tiling_on_sc=False, needs_layout_passes=False
        ),
    )
    return f(x2, fidx)


# speedup vs baseline: 2.1917x; 1.2968x over previous
import numpy as np
import jax
import jax.numpy as jnp
from jax import lax
from jax.experimental import pallas as pl
from jax.experimental.pallas import tpu as pltpu
from jax.experimental.pallas import tpu_sc as plsc

_B, _N = 4096, 128
_M = _N * _N
_K = _N * (_N + 1) // 2
_NT = _K // 16
_UNROLL = 12

_NW = 32
_BPW = _B // _NW

_ROW_IDX, _COL_IDX = np.triu_indices(_N)
_FLAT_IDX = (_ROW_IDX * _N + _COL_IDX).astype(np.int32)

_SEG_OFF = np.concatenate([[0], np.cumsum(np.arange(_N, 0, -1))])
_TILE_PLAN = []
for _t in range(_NT):
    _lo = 16 * _t
    _i = int(np.searchsorted(_SEG_OFF, _lo, side="right") - 1)
    if _SEG_OFF[_i + 1] >= _lo + 16:
        _TILE_PLAN.append((_lo, _i * (_N + 1) + (_lo - int(_SEG_OFF[_i]))))
    else:
        _TILE_PLAN.append((_lo, None))


def _sc_body(x_hbm, idx_hbm, out_hbm, idx_v, in0, in1, ou0, ou1,
             is0, is1, os0, os1):
    c = lax.axis_index("c")
    s = lax.axis_index("s")
    wid = s * 2 + c
    b0 = wid * _BPW

    pltpu.sync_copy(idx_hbm, idx_v)

    bufs = ((in0, ou0, is0, os0), (in1, ou1, is1, os1))

    def start_in(p, b):
        iv, _, isem, _ = bufs[p]
        pltpu.async_copy(x_hbm.at[b], iv, isem)

    def wait_in(p, b):
        iv, _, isem, _ = bufs[p]
        pltpu.make_async_copy(x_hbm.at[b], iv, isem).wait()

    def start_out(p, b):
        _, ov, _, osem = bufs[p]
        pltpu.async_copy(ov, out_hbm.at[b], osem)

    def wait_out(p, b):
        _, ov, _, osem = bufs[p]
        pltpu.make_async_copy(ov, out_hbm.at[b], osem).wait()

    start_in(0, b0)
    start_in(1, b0 + 1)

    def pair(rr, carry):
        for p in (0, 1):
            r = rr * 2 + p
            b = b0 + r
            iv, ov, _, _ = bufs[p]
            wait_in(p, b)

            @pl.when(rr > 0)
            def _():
                wait_out(p, b - 2)

            for o, so in _TILE_PLAN:
                if so is not None:
                    ov[pl.ds(o, 16)] = iv[pl.ds(so, 16)]
                else:
                    idx = idx_v[pl.ds(o, 16)]
                    ov[pl.ds(o, 16)] = plsc.load_gather(iv, [idx])
            start_out(p, b)

            @pl.when(r + 2 < _BPW)
            def _():
                start_in(p, b + 2)
        return carry

    lax.fori_loop(0, _BPW // 2, pair, 0)

    wait_out(0, b0 + _BPW - 2)
    wait_out(1, b0 + _BPW - 1)


def kernel(input):
    x2 = input.reshape(_B, _M)
    fidx = jnp.asarray(_FLAT_IDX)
    mesh = plsc.VectorSubcoreMesh(core_axis_name="c", subcore_axis_name="s")
    f = pl.kernel(
        _sc_body,
        mesh=mesh,
        out_type=jax.ShapeDtypeStruct((_B, _K), jnp.float32),
        scratch_types=[
            pltpu.VMEM((_K,), jnp.int32),
            pltpu.VMEM((_M,), jnp.float32),
            pltpu.VMEM((_M,), jnp.float32),
            pltpu.VMEM((_K,), jnp.float32),
            pltpu.VMEM((_K,), jnp.float32),
            pltpu.SemaphoreType.DMA,
            pltpu.SemaphoreType.DMA,
            pltpu.SemaphoreType.DMA,
            pltpu.SemaphoreType.DMA,
        ],
        compiler_params=pltpu.CompilerParams(
            use_tc_tiling_on_sc=False, needs_layout_passes=False
        ),
    )
    return f(x2, fidx)
